# baseline (speedup 1.0000x reference)
import functools

import jax
import jax.numpy as jnp
from jax import lax
from jax.experimental import pallas as pl
from jax.experimental.pallas import tpu as pltpu
from jax.experimental.pallas import tpu_sc as plsc

B, L, E = 4096, 200, 64
N = B * L
NC, NS = 2, 16
NW = NC * NS
W = N // NW
TN = 256
IT = W // TN
RPW = W // 128

_mesh = plsc.VectorSubcoreMesh(core_axis_name="c", subcore_axis_name="s")


@functools.partial(
    pl.kernel,
    out_type=jax.ShapeDtypeStruct((1600, 32, 1024), jnp.float32),
    mesh=_mesh,
    scratch_types=[
        pltpu.VMEM((RPW, 128), jnp.int32),
        pltpu.VMEM((2, TN, E), jnp.float32),
        pltpu.VMEM((2, 8, 2, 1024), jnp.float32),
        [pltpu.SemaphoreType.DMA] * 2,
        [pltpu.SemaphoreType.DMA] * 2,
    ],
    compiler_params=pltpu.CompilerParams(
        use_tc_tiling_on_sc=False, needs_layout_passes=False
    ),
)
def _emb(idx_hbm, table_hbm, out_hbm, idx_v, rows_v, tr_v, gsems, ssems):
    wid = lax.axis_index("s") * NC + lax.axis_index("c")
    base_row = wid * RPW
    pltpu.sync_copy(idx_hbm.at[pl.ds(base_row, RPW)], idx_v)

    def start_gathers(k, p):
        pltpu.async_copy(
            table_hbm.at[idx_v.at[2 * k]], rows_v.at[p, pl.ds(0, 128)], gsems[p]
        )
        pltpu.async_copy(
            table_hbm.at[idx_v.at[2 * k + 1]],
            rows_v.at[p, pl.ds(128, 128)],
            gsems[p],
        )

    def wait_gathers(p):
        for h in range(2):
            pltpu.make_async_copy(
                table_hbm.at[idx_v.at[0]],
                rows_v.at[p, pl.ds(128 * h, 128)],
                gsems[p],
            ).wait()

    def out_slice(k):
        t0 = wid * W + k * TN
        ll = t0 >> 12
        c0 = (t0 >> 7) & 31
        return out_hbm.at[pl.ds(8 * ll, 8), pl.ds(c0, 2), :]

    def wait_store(p):
        pltpu.make_async_copy(
            tr_v.at[p], out_hbm.at[pl.ds(0, 8), pl.ds(0, 2), :], ssems[p]
        ).wait()

    iota16 = lax.iota(jnp.int32, 16)
    ridx = [[iota16 + (128 * cp + 16 * g) for g in range(8)] for cp in range(2)]

    start_gathers(0, 0)

    def item(k, p, q):
        @pl.when(k + 1 < IT)
        def _():
            start_gathers(k + 1, q)

        @pl.when(k >= 2)
        def _():
            wait_store(p)

        wait_gathers(p)
        rows_p = rows_v.at[p]
        tr_p = tr_v.at[p]

        def tbody(eb, carry):
            for s in range(8):
                e = 8 * eb + s
                cvec = jnp.broadcast_to(e, (16,)).astype(jnp.int32)
                for cp in range(2):
                    for g in range(8):
                        v = plsc.load_gather(rows_p, [ridx[cp][g], cvec])
                        tr_p[eb, cp, pl.ds(s * 128 + 16 * g, 16)] = v
            return carry

        lax.fori_loop(0, 8, tbody, 0)
        pltpu.async_copy(tr_p, out_slice(k), ssems[p])

    def body(g, carry):
        item(2 * g, 0, 1)
        item(2 * g + 1, 1, 0)
        return carry

    lax.fori_loop(0, IT // 2, body, 0)
    wait_store(0)
    wait_store(1)


def kernel(tok_ids, table):
    idx = tok_ids.T.reshape(RPW * NW, 128).astype(jnp.int32)
    out_t = _emb(idx, table)
    out = (
        out_t.reshape(L, 8, 32, 8, 128)
        .transpose(2, 4, 0, 1, 3)
        .reshape(B, L, E)
    )
    return out


# speedup vs baseline: 1.6009x; 1.6009x over previous
import functools

import jax
import jax.numpy as jnp
from jax import lax
from jax.experimental import pallas as pl
from jax.experimental.pallas import tpu as pltpu
from jax.experimental.pallas import tpu_sc as plsc

B, L, E = 4096, 200, 64
N = B * L
NC, NS = 2, 16
NW = NC * NS
W = N // NW
CH = 128
NCH = W // CH
M = 8
K = 4
NG = NCH // M

_mesh = plsc.VectorSubcoreMesh(core_axis_name="c", subcore_axis_name="s")


@functools.partial(
    pl.kernel,
    out_type=jax.ShapeDtypeStruct((N, E), jnp.float32),
    mesh=_mesh,
    scratch_types=[
        pltpu.VMEM((NCH, CH), jnp.int32),
        pltpu.VMEM((M, CH, E), jnp.float32),
        [pltpu.SemaphoreType.DMA] * M,
        [pltpu.SemaphoreType.DMA] * M,
    ],
    compiler_params=pltpu.CompilerParams(
        use_tc_tiling_on_sc=False, needs_layout_passes=False
    ),
)
def _emb(idx_hbm, table_hbm, out_hbm, idx_v, rows_v, gsems, ssems):
    wid = lax.axis_index("s") * NC + lax.axis_index("c")
    base_ch = wid * NCH
    pltpu.sync_copy(idx_hbm.at[pl.ds(base_ch, NCH)], idx_v)

    def gather(j, b):
        return pltpu.make_async_copy(
            table_hbm.at[idx_v.at[j]], rows_v.at[b], gsems[b]
        )

    def store(j, b):
        return pltpu.make_async_copy(
            rows_v.at[b], out_hbm.at[pl.ds((base_ch + j) * CH, CH)], ssems[b]
        )

    for b in range(K):
        gather(b, b).start()

    def body(g, carry):
        for b in range(M):
            j = g * M + b
            gather(j, b).wait()
            store(j, b).start()
            bn = (b + K) % M
            @pl.when(j >= K)
            def _():
                store(j - K, bn).wait()

            @pl.when(j + K < NCH)
            def _():
                gather(j + K, bn).start()
        return carry

    lax.fori_loop(0, NG, body, 0)

    for b in range(K):
        j = NCH - K + b
        store(j, j % M).wait()


def kernel(tok_ids, table):
    idx = tok_ids.T.reshape(NW * NCH, CH).astype(jnp.int32)
    out = _emb(idx, table)
    return out.reshape(L, B, E).transpose(1, 0, 2)
